# baseline (speedup 1.0000x reference)
import jax
import jax.numpy as jnp
from jax.experimental import pallas as pl
from jax.experimental.pallas import tpu as pltpu

B, N_A, D_Z, D_A = 256, 32, 128, 128
D = D_Z + D_A
E, K, N_HEADS = 8, 2, 4
FFN, HEAD_H, N_R = 1024, 512, 4
IN_DIM = N_A * D
HD = D // N_HEADS

G = 16
S = G * N_A
NP = B * K
NBS = -(-(NP + E * (G - 1)) // G)
S_SLOTS = NBS * G


def _mmt(x, w):
    return jax.lax.dot_general(x, w, (((1,), (1,)), ((), ())),
                               preferred_element_type=jnp.float32)


def _mm(x, w):
    return jax.lax.dot_general(x, w, (((1,), (0,)), ((), ())),
                               preferred_element_type=jnp.float32)


def _bmmt(x, w):
    return jax.lax.dot_general(x.astype(jnp.bfloat16), w.astype(jnp.bfloat16),
                               (((1,), (1,)), ((), ())),
                               preferred_element_type=jnp.float32)


def _bmm(x, w):
    return jax.lax.dot_general(x.astype(jnp.bfloat16), w.astype(jnp.bfloat16),
                               (((1,), (0,)), ((), ())),
                               preferred_element_type=jnp.float32)


def _ln(x, w, b):
    m = jnp.mean(x, axis=-1, keepdims=True)
    xc = x - m
    v = jnp.mean(xc * xc, axis=-1, keepdims=True)
    return xc / jnp.sqrt(v + 1e-5) * w + b


def _routing_kernel(xf_ref, wg_ref, logits_ref, gates_ref, loss_ref,
                    bexp_ref, stok_ref, sgate_ref, bused_ref):
    xf = xf_ref[...]
    wg = wg_ref[...]
    logits = _mm(xf, wg)
    logits_ref[...] = logits
    cols = jax.lax.broadcasted_iota(jnp.int32, (B, E), 1)
    m1 = jnp.max(logits, axis=1, keepdims=True)
    ch1 = cols == jnp.min(jnp.where(logits == m1, cols, E), axis=1,
                          keepdims=True)
    l2 = jnp.where(ch1, -jnp.inf, logits)
    m2 = jnp.max(l2, axis=1, keepdims=True)
    ch2 = cols == jnp.min(jnp.where(l2 == m2, cols, E), axis=1, keepdims=True)
    e2 = jnp.exp(m2 - m1)
    g1 = 1.0 / (1.0 + e2)
    g2 = e2 / (1.0 + e2)
    gates = jnp.where(ch1, g1, 0.0) + jnp.where(ch2, g2, 0.0)
    gates_ref[...] = gates
    importance = jnp.sum(gates, axis=0)
    load = jnp.sum((gates > 0).astype(jnp.float32), axis=0)
    z_loss = jnp.mean(jnp.log(jnp.sum(jnp.exp(logits), axis=1)))

    def cv2(v):
        mu = jnp.mean(v)
        var = jnp.sum((v - mu) ** 2) / (E - 1)
        return var / (mu * mu + 1e-10)

    loss_ref[...] = jnp.reshape(cv2(importance) + cv2(load) + z_loss, (1, 1))

    oh = jnp.concatenate([ch1, ch2], axis=0).astype(jnp.float32)
    tok_col = jnp.concatenate(
        [jax.lax.broadcasted_iota(jnp.int32, (B, 1), 0).astype(jnp.float32)]
        * 2, axis=0)
    gate_col = jnp.concatenate([g1, g2], axis=0)
    counts = jnp.sum(oh, axis=0, keepdims=True)
    pc = jnp.ceil(counts / G) * G
    ei = jax.lax.broadcasted_iota(jnp.int32, (E, E), 0)
    ej = jax.lax.broadcasted_iota(jnp.int32, (E, E), 1)
    upper = (ei < ej).astype(jnp.float32)
    offs = _mm(pc, upper)
    pi = jax.lax.broadcasted_iota(jnp.int32, (NP, NP), 0)
    pj = jax.lax.broadcasted_iota(jnp.int32, (NP, NP), 1)
    lower = (pi >= pj).astype(jnp.float32)
    csum = _mm(lower, oh)
    rank = jnp.sum(oh * csum, axis=1, keepdims=True) - 1.0
    slot = jnp.sum(oh * offs, axis=1, keepdims=True) + rank
    sl_id = jax.lax.broadcasted_iota(jnp.int32, (1, S_SLOTS),
                                     1).astype(jnp.float32)
    hit = (slot == sl_id).astype(jnp.float32)
    stok_ref[...] = jnp.sum(hit * tok_col, axis=0,
                            keepdims=True).astype(jnp.int32)
    sgate_ref[...] = jnp.sum(hit * gate_col, axis=0, keepdims=True)
    bs = jax.lax.broadcasted_iota(jnp.int32, (NBS, 1),
                                  0).astype(jnp.float32) * G
    bexp_ref[...] = (jnp.sum((bs >= offs).astype(jnp.float32), axis=1,
                             keepdims=True) - 1.0).astype(jnp.int32)
    bused_ref[...] = (bs < jnp.sum(pc)).astype(jnp.int32)


def _experts_kernel(bexp_sref, stok_sref, bused_sref, x_ref, sgate_ref,
                    w_in_ref, b_in_ref, w_out_ref, b_out_ref, ln1w_ref,
                    ln1b_ref, w1_ref, b1_ref, w2_ref, b2_ref, ln2w_ref,
                    ln2b_ref, y_ref, win_bf, wout_bf, w1_bf, w2_bf):
    i = pl.program_id(0)

    @pl.when(i == 0)
    def _():
        y_ref[...] = jnp.zeros_like(y_ref)

    @pl.when(bused_sref[i] != 0)
    def _run_block():
        _expert_block(bexp_sref, stok_sref, x_ref, sgate_ref, w_in_ref,
                      b_in_ref, w_out_ref, b_out_ref, ln1w_ref, ln1b_ref,
                      w1_ref, b1_ref, w2_ref, b2_ref, ln2w_ref, ln2b_ref,
                      y_ref, win_bf, wout_bf, w1_bf, w2_bf, i)


def _expert_block(bexp_sref, stok_sref, x_ref, sgate_ref, w_in_ref,
                  b_in_ref, w_out_ref, b_out_ref, ln1w_ref, ln1b_ref,
                  w1_ref, b1_ref, w2_ref, b2_ref, ln2w_ref, ln2b_ref,
                  y_ref, win_bf, wout_bf, w1_bf, w2_bf, i):
    new_exp = jnp.logical_or(
        i == 0, bexp_sref[i] != bexp_sref[jnp.maximum(i - 1, 0)])

    @pl.when(new_exp)
    def _():
        win_bf[...] = w_in_ref[0].astype(jnp.bfloat16)
        wout_bf[...] = w_out_ref[0].astype(jnp.bfloat16)
        w1_bf[...] = w1_ref[0].astype(jnp.bfloat16)
        w2_bf[...] = w2_ref[0].astype(jnp.bfloat16)

    x = jnp.concatenate(
        [x_ref[pl.ds(stok_sref[i * G + g] * N_A, N_A), :] for g in range(G)],
        axis=0)
    qkv = _bmmt(x, win_bf[...]) + b_in_ref[0]
    q = qkv[:, :D]
    k = qkv[:, D:2 * D]
    v = qkv[:, 2 * D:]
    CR = 256
    row_t = jax.lax.broadcasted_iota(jnp.int32, (CR, CR), 0) // N_A
    col_t = jax.lax.broadcasted_iota(jnp.int32, (CR, CR), 1) // N_A
    same = row_t == col_t
    onecol = (jax.lax.broadcasted_iota(jnp.int32, (CR, HD), 1) == 0
              ).astype(jnp.float32)
    q = q * (1.0 / (HD ** 0.5))
    o_chunks = []
    for c in range(S // CR):
        avs = []
        for h in range(N_HEADS):
            qc = q[c * CR:(c + 1) * CR, h * HD:(h + 1) * HD]
            kc = k[c * CR:(c + 1) * CR, h * HD:(h + 1) * HD]
            vc = v[c * CR:(c + 1) * CR, h * HD:(h + 1) * HD]
            p = jnp.exp(jnp.where(same, _bmmt(qc, kc), -1e30))
            av = _bmm(p, jnp.concatenate([vc, onecol], axis=1))
            avs.append(av[:, :HD] / av[:, HD:HD + 1])
        o_chunks.append(jnp.concatenate(avs, axis=1))
    o = jnp.concatenate(o_chunks, axis=0)
    o = _bmmt(o, wout_bf[...]) + b_out_ref[0]
    hmid = _ln(x + o, ln1w_ref[0], ln1b_ref[0])
    f = jnp.maximum(_bmmt(hmid, w1_bf[...]) + b1_ref[0], 0.0)
    f = _bmmt(f, w2_bf[...]) + b2_ref[0]
    out = _ln(hmid + f, ln2w_ref[0], ln2b_ref[0])
    gsel = sgate_ref[pl.ds(i * G, G), :]
    rt = jax.lax.broadcasted_iota(jnp.int32, (S, G), 0) // N_A
    ct = jax.lax.broadcasted_iota(jnp.int32, (S, G), 1)
    g_rows = jnp.sum(jnp.where(rt == ct, gsel.T, 0.0), axis=1, keepdims=True)
    contrib = out * g_rows
    for g in range(G):
        rows = pl.ds(stok_sref[i * G + g] * N_A, N_A)
        y_ref[rows, :] = y_ref[rows, :] + contrib[g * N_A:(g + 1) * N_A, :]


def _head_kernel(y_ref, hw1_ref, hb1_ref, hw2_ref, hb2_ref, out_ref):
    h = jnp.maximum(_bmmt(y_ref[...], hw1_ref[...]) + hb1_ref[...], 0.0)
    out_ref[...] = _mmt(h, hw2_ref[...]) + hb2_ref[...]


def kernel(z, a, w_gate, ew_in, eb_in, ew_out, eb_out, eln1w, eln1b, ew1,
           eb1, ew2, eb2, eln2w, eln2b, hw1, hb1, hw2, hb2):
    x = jnp.concatenate([z, a], axis=-1)
    xf = x.reshape(B, IN_DIM)
    x2 = x.reshape(B * N_A, D)

    logits, gates, loss2, bexp, stok, sgate, bused = pl.pallas_call(
        _routing_kernel,
        out_shape=(
            jax.ShapeDtypeStruct((B, E), jnp.float32),
            jax.ShapeDtypeStruct((B, E), jnp.float32),
            jax.ShapeDtypeStruct((1, 1), jnp.float32),
            jax.ShapeDtypeStruct((NBS, 1), jnp.int32),
            jax.ShapeDtypeStruct((1, S_SLOTS), jnp.int32),
            jax.ShapeDtypeStruct((1, S_SLOTS), jnp.float32),
            jax.ShapeDtypeStruct((NBS, 1), jnp.int32),
        ),
    )(xf, w_gate)

    wspec = lambda shp: pl.BlockSpec((1,) + shp, lambda i, be, st, bu: (be[i], 0, 0))
    fix = lambda shp: pl.BlockSpec(shp, lambda i, be, st, bu: (0, 0))
    y = pl.pallas_call(
        _experts_kernel,
        grid_spec=pltpu.PrefetchScalarGridSpec(
            num_scalar_prefetch=3,
            grid=(NBS,),
            in_specs=[
                fix((B * N_A, D)),
                fix((S_SLOTS, 1)),
                wspec((3 * D, D)),
                wspec((1, 3 * D)),
                wspec((D, D)),
                wspec((1, D)),
                wspec((1, D)),
                wspec((1, D)),
                wspec((FFN, D)),
                wspec((1, FFN)),
                wspec((D, FFN)),
                wspec((1, D)),
                wspec((1, D)),
                wspec((1, D)),
            ],
            out_specs=pl.BlockSpec((B * N_A, D), lambda i, be, st, bu: (0, 0)),
            scratch_shapes=[
                pltpu.VMEM((3 * D, D), jnp.bfloat16),
                pltpu.VMEM((D, D), jnp.bfloat16),
                pltpu.VMEM((FFN, D), jnp.bfloat16),
                pltpu.VMEM((D, FFN), jnp.bfloat16),
            ],
        ),
        out_shape=jax.ShapeDtypeStruct((B * N_A, D), jnp.float32),
        compiler_params=pltpu.CompilerParams(
            dimension_semantics=("arbitrary",)),
    )(bexp.reshape(NBS), stok.reshape(S_SLOTS), bused.reshape(NBS), x2,
      sgate.reshape(S_SLOTS, 1),
      ew_in, eb_in.reshape(E, 1, 3 * D),
      ew_out, eb_out.reshape(E, 1, D),
      eln1w.reshape(E, 1, D), eln1b.reshape(E, 1, D),
      ew1, eb1.reshape(E, 1, FFN),
      ew2, eb2.reshape(E, 1, D),
      eln2w.reshape(E, 1, D), eln2b.reshape(E, 1, D))

    r_logits = pl.pallas_call(
        _head_kernel,
        out_shape=jax.ShapeDtypeStruct((B, N_R), jnp.float32),
    )(y.reshape(B, IN_DIM), hw1, hb1.reshape(1, HEAD_H), hw2,
      hb2.reshape(1, N_R))

    return r_logits, loss2[0, 0], gates, logits


# speedup vs baseline: 1.0783x; 1.0783x over previous
import jax
import jax.numpy as jnp
from jax.experimental import pallas as pl
from jax.experimental.pallas import tpu as pltpu

B, N_A, D_Z, D_A = 256, 32, 128, 128
D = D_Z + D_A
E, K, N_HEADS = 8, 2, 4
FFN, HEAD_H, N_R = 1024, 512, 4
IN_DIM = N_A * D
HD = D // N_HEADS

G = 32
S = G * N_A
NP = B * K
NBS = -(-(NP + E * (G - 1)) // G)
S_SLOTS = NBS * G


def _mmt(x, w):
    return jax.lax.dot_general(x, w, (((1,), (1,)), ((), ())),
                               preferred_element_type=jnp.float32)


def _mm(x, w):
    return jax.lax.dot_general(x, w, (((1,), (0,)), ((), ())),
                               preferred_element_type=jnp.float32)


def _bmmt(x, w):
    return jax.lax.dot_general(x.astype(jnp.bfloat16), w.astype(jnp.bfloat16),
                               (((1,), (1,)), ((), ())),
                               preferred_element_type=jnp.float32)


def _bmm(x, w):
    return jax.lax.dot_general(x.astype(jnp.bfloat16), w.astype(jnp.bfloat16),
                               (((1,), (0,)), ((), ())),
                               preferred_element_type=jnp.float32)


def _ln(x, w, b):
    m = jnp.mean(x, axis=-1, keepdims=True)
    xc = x - m
    v = jnp.mean(xc * xc, axis=-1, keepdims=True)
    return xc / jnp.sqrt(v + 1e-5) * w + b


def _routing_kernel(xf_ref, wg_ref, logits_ref, gates_ref, loss_ref,
                    bexp_ref, stok_ref, sgate_ref, bused_ref):
    xf = xf_ref[...]
    wg = wg_ref[...]
    logits = _mm(xf, wg)
    logits_ref[...] = logits
    cols = jax.lax.broadcasted_iota(jnp.int32, (B, E), 1)
    m1 = jnp.max(logits, axis=1, keepdims=True)
    ch1 = cols == jnp.min(jnp.where(logits == m1, cols, E), axis=1,
                          keepdims=True)
    l2 = jnp.where(ch1, -jnp.inf, logits)
    m2 = jnp.max(l2, axis=1, keepdims=True)
    ch2 = cols == jnp.min(jnp.where(l2 == m2, cols, E), axis=1, keepdims=True)
    e2 = jnp.exp(m2 - m1)
    g1 = 1.0 / (1.0 + e2)
    g2 = e2 / (1.0 + e2)
    gates = jnp.where(ch1, g1, 0.0) + jnp.where(ch2, g2, 0.0)
    gates_ref[...] = gates
    importance = jnp.sum(gates, axis=0)
    load = jnp.sum((gates > 0).astype(jnp.float32), axis=0)
    z_loss = jnp.mean(jnp.log(jnp.sum(jnp.exp(logits), axis=1)))

    def cv2(v):
        mu = jnp.mean(v)
        var = jnp.sum((v - mu) ** 2) / (E - 1)
        return var / (mu * mu + 1e-10)

    loss_ref[...] = jnp.reshape(cv2(importance) + cv2(load) + z_loss, (1, 1))

    oh = jnp.concatenate([ch1, ch2], axis=0).astype(jnp.float32)
    tok_col = jnp.concatenate(
        [jax.lax.broadcasted_iota(jnp.int32, (B, 1), 0).astype(jnp.float32)]
        * 2, axis=0)
    gate_col = jnp.concatenate([g1, g2], axis=0)
    counts = jnp.sum(oh, axis=0, keepdims=True)
    pc = jnp.ceil(counts / G) * G
    ei = jax.lax.broadcasted_iota(jnp.int32, (E, E), 0)
    ej = jax.lax.broadcasted_iota(jnp.int32, (E, E), 1)
    upper = (ei < ej).astype(jnp.float32)
    offs = _mm(pc, upper)
    pi = jax.lax.broadcasted_iota(jnp.int32, (NP, NP), 0)
    pj = jax.lax.broadcasted_iota(jnp.int32, (NP, NP), 1)
    lower = (pi >= pj).astype(jnp.float32)
    csum = _mm(lower, oh)
    rank = jnp.sum(oh * csum, axis=1, keepdims=True) - 1.0
    slot = jnp.sum(oh * offs, axis=1, keepdims=True) + rank
    sl_id = jax.lax.broadcasted_iota(jnp.int32, (1, S_SLOTS),
                                     1).astype(jnp.float32)
    hit = (slot == sl_id).astype(jnp.float32)
    stok_ref[...] = jnp.sum(hit * tok_col, axis=0,
                            keepdims=True).astype(jnp.int32)
    sgate_ref[...] = jnp.sum(hit * gate_col, axis=0, keepdims=True)
    bs = jax.lax.broadcasted_iota(jnp.int32, (NBS, 1),
                                  0).astype(jnp.float32) * G
    bexp_ref[...] = (jnp.sum((bs >= offs).astype(jnp.float32), axis=1,
                             keepdims=True) - 1.0).astype(jnp.int32)
    bused_ref[...] = (bs < jnp.sum(pc)).astype(jnp.int32)


def _experts_kernel(bexp_sref, stok_sref, bused_sref, x_ref, sgate_ref,
                    w_in_ref, b_in_ref, w_out_ref, b_out_ref, ln1w_ref,
                    ln1b_ref, w1_ref, b1_ref, w2_ref, b2_ref, ln2w_ref,
                    ln2b_ref, y_ref, win_bf, wout_bf, w1_bf, w2_bf):
    i = pl.program_id(0)

    @pl.when(i == 0)
    def _():
        y_ref[...] = jnp.zeros_like(y_ref)

    @pl.when(bused_sref[i] != 0)
    def _run_block():
        _expert_block(bexp_sref, stok_sref, x_ref, sgate_ref, w_in_ref,
                      b_in_ref, w_out_ref, b_out_ref, ln1w_ref, ln1b_ref,
                      w1_ref, b1_ref, w2_ref, b2_ref, ln2w_ref, ln2b_ref,
                      y_ref, win_bf, wout_bf, w1_bf, w2_bf, i)


def _expert_block(bexp_sref, stok_sref, x_ref, sgate_ref, w_in_ref,
                  b_in_ref, w_out_ref, b_out_ref, ln1w_ref, ln1b_ref,
                  w1_ref, b1_ref, w2_ref, b2_ref, ln2w_ref, ln2b_ref,
                  y_ref, win_bf, wout_bf, w1_bf, w2_bf, i):
    new_exp = jnp.logical_or(
        i == 0, bexp_sref[i] != bexp_sref[jnp.maximum(i - 1, 0)])

    @pl.when(new_exp)
    def _():
        win_bf[...] = w_in_ref[0].astype(jnp.bfloat16)
        wout_bf[...] = w_out_ref[0].astype(jnp.bfloat16)
        w1_bf[...] = w1_ref[0].astype(jnp.bfloat16)
        w2_bf[...] = w2_ref[0].astype(jnp.bfloat16)

    x = jnp.concatenate(
        [x_ref[pl.ds(stok_sref[i * G + g] * N_A, N_A), :] for g in range(G)],
        axis=0)
    qkv = _bmmt(x, win_bf[...]) + b_in_ref[0]
    q = qkv[:, :D]
    k = qkv[:, D:2 * D]
    v = qkv[:, 2 * D:]
    CR = 256
    row_t = jax.lax.broadcasted_iota(jnp.int32, (CR, CR), 0) // N_A
    col_t = jax.lax.broadcasted_iota(jnp.int32, (CR, CR), 1) // N_A
    same = row_t == col_t
    onecol = (jax.lax.broadcasted_iota(jnp.int32, (CR, HD), 1) == 0
              ).astype(jnp.float32)
    q = q * (1.0 / (HD ** 0.5))
    o_chunks = []
    for c in range(S // CR):
        avs = []
        for h in range(N_HEADS):
            qc = q[c * CR:(c + 1) * CR, h * HD:(h + 1) * HD]
            kc = k[c * CR:(c + 1) * CR, h * HD:(h + 1) * HD]
            vc = v[c * CR:(c + 1) * CR, h * HD:(h + 1) * HD]
            p = jnp.exp(jnp.where(same, _bmmt(qc, kc), -1e30))
            av = _bmm(p, jnp.concatenate([vc, onecol], axis=1))
            avs.append(av[:, :HD] / av[:, HD:HD + 1])
        o_chunks.append(jnp.concatenate(avs, axis=1))
    o = jnp.concatenate(o_chunks, axis=0)
    o = _bmmt(o, wout_bf[...]) + b_out_ref[0]
    hmid = _ln(x + o, ln1w_ref[0], ln1b_ref[0])
    f = jnp.maximum(_bmmt(hmid, w1_bf[...]) + b1_ref[0], 0.0)
    f = _bmmt(f, w2_bf[...]) + b2_ref[0]
    out = _ln(hmid + f, ln2w_ref[0], ln2b_ref[0])
    gsel = sgate_ref[pl.ds(i * G, G), :]
    rt = jax.lax.broadcasted_iota(jnp.int32, (S, G), 0) // N_A
    ct = jax.lax.broadcasted_iota(jnp.int32, (S, G), 1)
    g_rows = jnp.sum(jnp.where(rt == ct, gsel.T, 0.0), axis=1, keepdims=True)
    contrib = out * g_rows
    for g in range(G):
        rows = pl.ds(stok_sref[i * G + g] * N_A, N_A)
        y_ref[rows, :] = y_ref[rows, :] + contrib[g * N_A:(g + 1) * N_A, :]


def _head_kernel(y_ref, hw1_ref, hb1_ref, hw2_ref, hb2_ref, out_ref):
    h = jnp.maximum(_mmt(y_ref[...], hw1_ref[...]) + hb1_ref[...], 0.0)
    out_ref[...] = _mmt(h, hw2_ref[...]) + hb2_ref[...]


def kernel(z, a, w_gate, ew_in, eb_in, ew_out, eb_out, eln1w, eln1b, ew1,
           eb1, ew2, eb2, eln2w, eln2b, hw1, hb1, hw2, hb2):
    x = jnp.concatenate([z, a], axis=-1)
    xf = x.reshape(B, IN_DIM)
    x2 = x.reshape(B * N_A, D)

    logits, gates, loss2, bexp, stok, sgate, bused = pl.pallas_call(
        _routing_kernel,
        out_shape=(
            jax.ShapeDtypeStruct((B, E), jnp.float32),
            jax.ShapeDtypeStruct((B, E), jnp.float32),
            jax.ShapeDtypeStruct((1, 1), jnp.float32),
            jax.ShapeDtypeStruct((NBS, 1), jnp.int32),
            jax.ShapeDtypeStruct((1, S_SLOTS), jnp.int32),
            jax.ShapeDtypeStruct((1, S_SLOTS), jnp.float32),
            jax.ShapeDtypeStruct((NBS, 1), jnp.int32),
        ),
    )(xf, w_gate)

    wspec = lambda shp: pl.BlockSpec((1,) + shp, lambda i, be, st, bu: (be[i], 0, 0))
    fix = lambda shp: pl.BlockSpec(shp, lambda i, be, st, bu: (0, 0))
    y = pl.pallas_call(
        _experts_kernel,
        grid_spec=pltpu.PrefetchScalarGridSpec(
            num_scalar_prefetch=3,
            grid=(NBS,),
            in_specs=[
                fix((B * N_A, D)),
                fix((S_SLOTS, 1)),
                wspec((3 * D, D)),
                wspec((1, 3 * D)),
                wspec((D, D)),
                wspec((1, D)),
                wspec((1, D)),
                wspec((1, D)),
                wspec((FFN, D)),
                wspec((1, FFN)),
                wspec((D, FFN)),
                wspec((1, D)),
                wspec((1, D)),
                wspec((1, D)),
            ],
            out_specs=pl.BlockSpec((B * N_A, D), lambda i, be, st, bu: (0, 0)),
            scratch_shapes=[
                pltpu.VMEM((3 * D, D), jnp.bfloat16),
                pltpu.VMEM((D, D), jnp.bfloat16),
                pltpu.VMEM((FFN, D), jnp.bfloat16),
                pltpu.VMEM((D, FFN), jnp.bfloat16),
            ],
        ),
        out_shape=jax.ShapeDtypeStruct((B * N_A, D), jnp.float32),
        compiler_params=pltpu.CompilerParams(
            dimension_semantics=("arbitrary",)),
    )(bexp.reshape(NBS), stok.reshape(S_SLOTS), bused.reshape(NBS), x2,
      sgate.reshape(S_SLOTS, 1),
      ew_in, eb_in.reshape(E, 1, 3 * D),
      ew_out, eb_out.reshape(E, 1, D),
      eln1w.reshape(E, 1, D), eln1b.reshape(E, 1, D),
      ew1, eb1.reshape(E, 1, FFN),
      ew2, eb2.reshape(E, 1, D),
      eln2w.reshape(E, 1, D), eln2b.reshape(E, 1, D))

    r_logits = pl.pallas_call(
        _head_kernel,
        out_shape=jax.ShapeDtypeStruct((B, N_R), jnp.float32),
    )(y.reshape(B, IN_DIM), hw1, hb1.reshape(1, HEAD_H), hw2,
      hb2.reshape(1, N_R))

    return r_logits, loss2[0, 0], gates, logits
